# baseline (device time: 79516 ns/iter reference)
import jax
import jax.numpy as jnp
from jax import lax
from jax.experimental import pallas as pl
from jax.experimental.pallas import tpu as pltpu

M = 4096
N_OUT = 1024
HALF = M // 2
CHUNKS = 32
RC = HALF // CHUNKS


def kernel(x):
    def body(x_ref, out_ref, send_buf, recv_buf, s1, r1, s2, r2):
        my_x = lax.axis_index("x")
        my_y = lax.axis_index("y")
        y_nbr = (my_x, 1 - my_y)
        x_nbr = (1 - my_x, my_y)

        barrier_sem = pltpu.get_barrier_semaphore()
        for nbr in (y_nbr, x_nbr):
            pl.semaphore_signal(
                barrier_sem, inc=1,
                device_id=nbr, device_id_type=pl.DeviceIdType.MESH,
            )
        pl.semaphore_wait(barrier_sem, 2)

        row0 = my_x * HALF
        my_col0 = my_y * N_OUT
        nbr_col0 = (1 - my_y) * N_OUT

        def stage(k):
            send_buf[pl.ds(k * RC, RC), :] = x_ref[
                0, pl.ds(row0 + k * RC, RC), pl.ds(nbr_col0, N_OUT)
            ].astype(jnp.bfloat16)

        def send_y(k):
            rdma = pltpu.make_async_remote_copy(
                src_ref=send_buf.at[pl.ds(k * RC, RC), :],
                dst_ref=recv_buf.at[pl.ds(k * RC, RC), :],
                send_sem=s1.at[k],
                recv_sem=r1.at[k],
                device_id=y_nbr,
                device_id_type=pl.DeviceIdType.MESH,
            )
            rdma.start()
            return rdma

        stage(0)
        rdmas1 = [send_y(0)]
        rdmas2 = []
        for k in range(CHUNKS):
            if k + 1 < CHUNKS:
                stage(k + 1)
                rdmas1.append(send_y(k + 1))

            rdmas1[k].wait_recv()
            local = x_ref[
                0, pl.ds(row0 + k * RC, RC), pl.ds(my_col0, N_OUT)
            ].astype(jnp.bfloat16)
            out_ref[pl.ds(row0 + k * RC, RC), :] = local + recv_buf[
                pl.ds(k * RC, RC), :
            ]

            rdma2 = pltpu.make_async_remote_copy(
                src_ref=out_ref.at[pl.ds(row0 + k * RC, RC), :],
                dst_ref=out_ref.at[pl.ds(row0 + k * RC, RC), :],
                send_sem=s2.at[k],
                recv_sem=r2.at[k],
                device_id=x_nbr,
                device_id_type=pl.DeviceIdType.MESH,
            )
            rdma2.start()
            rdmas2.append(rdma2)

        for k in range(CHUNKS):
            rdmas1[k].wait_send()
            rdmas2[k].wait_send()
            rdmas2[k].wait_recv()

    return pl.pallas_call(
        body,
        out_shape=jax.ShapeDtypeStruct((M, N_OUT), jnp.bfloat16),
        in_specs=[pl.BlockSpec(memory_space=pltpu.VMEM)],
        out_specs=pl.BlockSpec(memory_space=pltpu.VMEM),
        scratch_shapes=[
            pltpu.VMEM((HALF, N_OUT), jnp.bfloat16),
            pltpu.VMEM((HALF, N_OUT), jnp.bfloat16),
            pltpu.SemaphoreType.DMA((CHUNKS,)),
            pltpu.SemaphoreType.DMA((CHUNKS,)),
            pltpu.SemaphoreType.DMA((CHUNKS,)),
            pltpu.SemaphoreType.DMA((CHUNKS,)),
        ],
        compiler_params=pltpu.CompilerParams(
            collective_id=0,
            vmem_limit_bytes=64 * 1024 * 1024,
        ),
    )(x)


# device time: 75430 ns/iter; 1.0542x vs baseline; 1.0542x over previous
import jax
import jax.numpy as jnp
from jax import lax
from jax.experimental import pallas as pl
from jax.experimental.pallas import tpu as pltpu

M = 4096
N_TOT = 2048
N_OUT = 1024
HALF = M // 2
CHUNKS = 16
RC = HALF // CHUNKS


def kernel(x):
    def body(x_ref, out_ref, cast_buf, recv_buf, s1, r1, s2, r2):
        my_x = lax.axis_index("x")
        my_y = lax.axis_index("y")
        y_nbr = (my_x, 1 - my_y)
        x_nbr = (1 - my_x, my_y)

        barrier_sem = pltpu.get_barrier_semaphore()
        for nbr in (y_nbr, x_nbr):
            pl.semaphore_signal(
                barrier_sem, inc=1,
                device_id=nbr, device_id_type=pl.DeviceIdType.MESH,
            )
        pl.semaphore_wait(barrier_sem, 2)

        row0 = my_x * HALF
        my_col0 = my_y * N_OUT
        nbr_col0 = (1 - my_y) * N_OUT

        def cast(k):
            cast_buf[pl.ds(k * RC, RC), :] = x_ref[
                0, pl.ds(row0 + k * RC, RC), :
            ].astype(jnp.bfloat16)

        def send_y(k):
            rdma = pltpu.make_async_remote_copy(
                src_ref=cast_buf.at[pl.ds(k * RC, RC), pl.ds(nbr_col0, N_OUT)],
                dst_ref=recv_buf.at[pl.ds(k * RC, RC), :],
                send_sem=s1.at[k],
                recv_sem=r1.at[k],
                device_id=y_nbr,
                device_id_type=pl.DeviceIdType.MESH,
            )
            rdma.start()
            return rdma

        cast(0)
        rdmas1 = [send_y(0)]
        rdmas2 = []
        for k in range(CHUNKS):
            if k + 1 < CHUNKS:
                cast(k + 1)
                rdmas1.append(send_y(k + 1))

            rdmas1[k].wait_recv()
            out_ref[pl.ds(row0 + k * RC, RC), :] = (
                cast_buf[pl.ds(k * RC, RC), pl.ds(my_col0, N_OUT)]
                + recv_buf[pl.ds(k * RC, RC), :]
            )

            rdma2 = pltpu.make_async_remote_copy(
                src_ref=out_ref.at[pl.ds(row0 + k * RC, RC), :],
                dst_ref=out_ref.at[pl.ds(row0 + k * RC, RC), :],
                send_sem=s2.at[k],
                recv_sem=r2.at[k],
                device_id=x_nbr,
                device_id_type=pl.DeviceIdType.MESH,
            )
            rdma2.start()
            rdmas2.append(rdma2)

        for k in range(CHUNKS):
            rdmas1[k].wait_send()
            rdmas2[k].wait_send()
            rdmas2[k].wait_recv()

    return pl.pallas_call(
        body,
        out_shape=jax.ShapeDtypeStruct((M, N_OUT), jnp.bfloat16),
        in_specs=[pl.BlockSpec(memory_space=pltpu.VMEM)],
        out_specs=pl.BlockSpec(memory_space=pltpu.VMEM),
        scratch_shapes=[
            pltpu.VMEM((HALF, N_TOT), jnp.bfloat16),
            pltpu.VMEM((HALF, N_OUT), jnp.bfloat16),
            pltpu.SemaphoreType.DMA((CHUNKS,)),
            pltpu.SemaphoreType.DMA((CHUNKS,)),
            pltpu.SemaphoreType.DMA((CHUNKS,)),
            pltpu.SemaphoreType.DMA((CHUNKS,)),
        ],
        compiler_params=pltpu.CompilerParams(
            collective_id=0,
            vmem_limit_bytes=64 * 1024 * 1024,
        ),
    )(x)


# device time: 66050 ns/iter; 1.2039x vs baseline; 1.1420x over previous
import jax
import jax.numpy as jnp
from jax import lax
from jax.experimental import pallas as pl
from jax.experimental.pallas import tpu as pltpu

M = 4096
N_TOT = 2048
N_OUT = 1024
HALF = M // 2
CHUNKS = 16
RC = HALF // CHUNKS


def kernel(x):
    def body(x_ref, out_ref, xv_buf, cast_buf, recv_buf, ld, s1, r1, s2, r2):
        my_x = lax.axis_index("x")
        my_y = lax.axis_index("y")
        y_nbr = (my_x, 1 - my_y)
        x_nbr = (1 - my_x, my_y)

        row0 = my_x * HALF
        my_col0 = my_y * N_OUT
        nbr_col0 = (1 - my_y) * N_OUT

        loads = []
        for k in range(CHUNKS):
            cp = pltpu.make_async_copy(
                x_ref.at[0, pl.ds(row0 + k * RC, RC), :],
                xv_buf.at[pl.ds(k * RC, RC), :],
                ld.at[k],
            )
            cp.start()
            loads.append(cp)

        barrier_sem = pltpu.get_barrier_semaphore()
        for nbr in (y_nbr, x_nbr):
            pl.semaphore_signal(
                barrier_sem, inc=1,
                device_id=nbr, device_id_type=pl.DeviceIdType.MESH,
            )
        pl.semaphore_wait(barrier_sem, 2)

        def cast(k):
            loads[k].wait()
            cast_buf[pl.ds(k * RC, RC), :] = xv_buf[
                pl.ds(k * RC, RC), :
            ].astype(jnp.bfloat16)

        def send_y(k):
            rdma = pltpu.make_async_remote_copy(
                src_ref=cast_buf.at[pl.ds(k * RC, RC), pl.ds(nbr_col0, N_OUT)],
                dst_ref=recv_buf.at[pl.ds(k * RC, RC), :],
                send_sem=s1.at[k],
                recv_sem=r1.at[k],
                device_id=y_nbr,
                device_id_type=pl.DeviceIdType.MESH,
            )
            rdma.start()
            return rdma

        cast(0)
        rdmas1 = [send_y(0)]
        rdmas2 = []
        for k in range(CHUNKS):
            if k + 1 < CHUNKS:
                cast(k + 1)
                rdmas1.append(send_y(k + 1))

            rdmas1[k].wait_recv()
            out_ref[pl.ds(row0 + k * RC, RC), :] = (
                cast_buf[pl.ds(k * RC, RC), pl.ds(my_col0, N_OUT)]
                + recv_buf[pl.ds(k * RC, RC), :]
            )

            rdma2 = pltpu.make_async_remote_copy(
                src_ref=out_ref.at[pl.ds(row0 + k * RC, RC), :],
                dst_ref=out_ref.at[pl.ds(row0 + k * RC, RC), :],
                send_sem=s2.at[k],
                recv_sem=r2.at[k],
                device_id=x_nbr,
                device_id_type=pl.DeviceIdType.MESH,
            )
            rdma2.start()
            rdmas2.append(rdma2)

        for k in range(CHUNKS):
            rdmas1[k].wait_send()
            rdmas2[k].wait_send()
            rdmas2[k].wait_recv()

    return pl.pallas_call(
        body,
        out_shape=jax.ShapeDtypeStruct((M, N_OUT), jnp.bfloat16),
        in_specs=[pl.BlockSpec(memory_space=pl.ANY)],
        out_specs=pl.BlockSpec(memory_space=pltpu.VMEM),
        scratch_shapes=[
            pltpu.VMEM((HALF, N_TOT), jnp.float32),
            pltpu.VMEM((HALF, N_TOT), jnp.bfloat16),
            pltpu.VMEM((HALF, N_OUT), jnp.bfloat16),
            pltpu.SemaphoreType.DMA((CHUNKS,)),
            pltpu.SemaphoreType.DMA((CHUNKS,)),
            pltpu.SemaphoreType.DMA((CHUNKS,)),
            pltpu.SemaphoreType.DMA((CHUNKS,)),
            pltpu.SemaphoreType.DMA((CHUNKS,)),
        ],
        compiler_params=pltpu.CompilerParams(
            collective_id=0,
            vmem_limit_bytes=64 * 1024 * 1024,
        ),
    )(x)


# device time: 63260 ns/iter; 1.2570x vs baseline; 1.0441x over previous
import jax
import jax.numpy as jnp
from jax import lax
from jax.experimental import pallas as pl
from jax.experimental.pallas import tpu as pltpu

M = 4096
N_TOT = 2048
N_OUT = 1024
HALF = M // 2
CHUNKS = 16
RC = HALF // CHUNKS


def kernel(x):
    def body(x_ref, out_ref, xv, stage, recv, red,
             ld, s1, r1, s2, r2, st):
        my_x = lax.axis_index("x")
        my_y = lax.axis_index("y")
        y_nbr = (my_x, 1 - my_y)
        x_nbr = (1 - my_x, my_y)

        row0 = my_x * HALF
        my_col0 = my_y * N_OUT
        nbr_col0 = (1 - my_y) * N_OUT

        loads = []
        for k in range(CHUNKS):
            cp = pltpu.make_async_copy(
                x_ref.at[0, pl.ds(row0 + k * RC, RC), :],
                xv.at[pl.ds(k * RC, RC), :],
                ld.at[k],
            )
            cp.start()
            loads.append(cp)

        barrier_sem = pltpu.get_barrier_semaphore()
        for nbr in (y_nbr, x_nbr):
            pl.semaphore_signal(
                barrier_sem, inc=1,
                device_id=nbr, device_id_type=pl.DeviceIdType.MESH,
            )
        pl.semaphore_wait(barrier_sem, 2)

        rdmas1 = []

        def stage_and_send(k):
            loads[k].wait()
            stage[pl.ds(k * RC, RC), :] = xv[
                pl.ds(k * RC, RC), pl.ds(nbr_col0, N_OUT)
            ].astype(jnp.bfloat16)
            rdma = pltpu.make_async_remote_copy(
                src_ref=stage.at[pl.ds(k * RC, RC), :],
                dst_ref=recv.at[pl.ds(k * RC, RC), :],
                send_sem=s1.at[k],
                recv_sem=r1.at[k],
                device_id=y_nbr,
                device_id_type=pl.DeviceIdType.MESH,
            )
            rdma.start()
            rdmas1.append(rdma)

        stage_and_send(0)
        stage_and_send(1)
        rdmas2 = []
        stores = []
        for k in range(CHUNKS):
            if k + 2 < CHUNKS:
                stage_and_send(k + 2)

            rdmas1[k].wait_recv()
            red[pl.ds(k * RC, RC), :] = (
                xv[pl.ds(k * RC, RC), pl.ds(my_col0, N_OUT)]
                + recv[pl.ds(k * RC, RC), :].astype(jnp.float32)
            ).astype(jnp.bfloat16)

            rdma2 = pltpu.make_async_remote_copy(
                src_ref=red.at[pl.ds(k * RC, RC), :],
                dst_ref=out_ref.at[pl.ds(row0 + k * RC, RC), :],
                send_sem=s2.at[k],
                recv_sem=r2.at[k],
                device_id=x_nbr,
                device_id_type=pl.DeviceIdType.MESH,
            )
            rdma2.start()
            rdmas2.append(rdma2)
            cp = pltpu.make_async_copy(
                red.at[pl.ds(k * RC, RC), :],
                out_ref.at[pl.ds(row0 + k * RC, RC), :],
                st.at[k],
            )
            cp.start()
            stores.append(cp)

        for k in range(CHUNKS):
            rdmas1[k].wait_send()
            rdmas2[k].wait_send()
            rdmas2[k].wait_recv()
            stores[k].wait()

    return pl.pallas_call(
        body,
        out_shape=jax.ShapeDtypeStruct((M, N_OUT), jnp.bfloat16),
        in_specs=[pl.BlockSpec(memory_space=pl.ANY)],
        out_specs=pl.BlockSpec(memory_space=pl.ANY),
        scratch_shapes=[
            pltpu.VMEM((HALF, N_TOT), jnp.float32),
            pltpu.VMEM((HALF, N_OUT), jnp.bfloat16),
            pltpu.VMEM((HALF, N_OUT), jnp.bfloat16),
            pltpu.VMEM((HALF, N_OUT), jnp.bfloat16),
            pltpu.SemaphoreType.DMA((CHUNKS,)),
            pltpu.SemaphoreType.DMA((CHUNKS,)),
            pltpu.SemaphoreType.DMA((CHUNKS,)),
            pltpu.SemaphoreType.DMA((CHUNKS,)),
            pltpu.SemaphoreType.DMA((CHUNKS,)),
            pltpu.SemaphoreType.DMA((CHUNKS,)),
        ],
        compiler_params=pltpu.CompilerParams(
            collective_id=0,
            vmem_limit_bytes=64 * 1024 * 1024,
        ),
    )(x)
